# Initial kernel scaffold; baseline (speedup 1.0000x reference)
import jax
import jax.numpy as jnp
from jax.experimental import pallas as pl


def kernel(x, token_table, pos_table):
    raise NotImplementedError("write your pallas kernel here")


# speedup vs baseline: 1.0107x; 1.0107x over previous
import functools
import math

import jax
import jax.numpy as jnp
from jax import lax
from jax.experimental import pallas as pl
from jax.experimental.pallas import tpu as pltpu
from jax.experimental.pallas import tpu_sc as plsc

_LANES = 16


def kernel(x, token_table, pos_table):
    B, S = x.shape
    V, D = token_table.shape
    T = B * S
    scale = math.sqrt(float(D))

    info = plsc.get_sparse_core_info()
    NC, NS = info.num_cores, info.num_subcores
    NW = NC * NS
    CB = 64
    n_chunks = T // (NW * CB)

    x2 = x.reshape(T // CB, CB).astype(jnp.int32)
    mesh = plsc.VectorSubcoreMesh(core_axis_name="c", subcore_axis_name="s")

    @functools.partial(
        pl.kernel,
        mesh=mesh,
        out_type=jax.ShapeDtypeStruct((T, D), jnp.float32),
        scratch_types=[
            pltpu.VMEM((n_chunks, CB), jnp.int32),
            pltpu.VMEM((CB, D), jnp.float32),
            pltpu.VMEM((CB, D), jnp.float32),
            pltpu.SemaphoreType.DMA,
        ],
    )
    def emb_kernel(x_hbm, tok_hbm, pos_hbm, out_hbm, idx_v, rows_v, pos_v, sem):
        wid = lax.axis_index("s") * NC + lax.axis_index("c")
        base = wid * (n_chunks * CB)
        s0 = lax.rem(base, S)
        pltpu.sync_copy(x_hbm.at[pl.ds(wid * n_chunks, n_chunks)], idx_v)
        scale_v = jnp.full((_LANES,), scale, jnp.float32)
        for j in range(n_chunks):
            pltpu.async_copy(tok_hbm.at[idx_v.at[j]], rows_v, sem).wait()
            pltpu.sync_copy(pos_hbm.at[pl.ds(s0 + j * CB, CB)], pos_v)

            def row_body(i, carry):
                for c in range(D // _LANES):
                    sl = pl.ds(c * _LANES, _LANES)
                    rows_v[i, sl] = rows_v[i, sl] * scale_v + pos_v[i, sl]
                return carry

            lax.fori_loop(0, CB, row_body, 0)
            pltpu.sync_copy(rows_v, out_hbm.at[pl.ds(base + j * CB, CB)])

    out = emb_kernel(x2, token_table, pos_table)
    return out.reshape(B, S, D)
